# Initial kernel scaffold; baseline (speedup 1.0000x reference)
import jax
import jax.numpy as jnp
from jax.experimental import pallas as pl


def kernel(x, tables):
    raise NotImplementedError("write your pallas kernel here")


# speedup vs baseline: 1.2037x; 1.2037x over previous
import functools

import jax
import jax.numpy as jnp
from jax import lax
from jax.experimental import pallas as pl
from jax.experimental.pallas import tpu as pltpu
from jax.experimental.pallas import tpu_sc as plsc

N_FIELDS = 26
VOCAB = 100000
EMBED = 32
BATCH = 16384
TOTAL = BATCH * N_FIELDS

_INFO = plsc.get_sparse_core_info()
_NC = _INFO.num_cores
_NS = _INFO.num_subcores
NW = _NC * _NS
PER_W = TOTAL // NW
NCHUNK = 13
CHUNK = PER_W // NCHUNK
LANES = 16


def _body(tab_hbm, x_hbm, out_hbm, idx_v, rows0, rows1,
          gsem0, gsem1, osem0, osem1):
    wid = lax.axis_index("s") * _NC + lax.axis_index("c")
    base = wid * PER_W

    pltpu.sync_copy(x_hbm.at[pl.ds(base, PER_W)], idx_v)

    rows = (rows0, rows1)
    gsem = (gsem0, gsem1)
    osem = (osem0, osem1)

    def compute(j):
        def step(k, carry):
            s = j * CHUNK + k * LANES
            pos = s + lax.broadcasted_iota(jnp.int32, (LANES,), 0)
            off = lax.rem(pos, N_FIELDS) * VOCAB
            idx_v[pl.ds(s, LANES)] = idx_v[pl.ds(s, LANES)] + off
            return carry
        lax.fori_loop(0, CHUNK // LANES, step, 0)

    def gather_start(j):
        b = j % 2
        cp = pltpu.make_async_copy(
            tab_hbm.at[idx_v.at[pl.ds(j * CHUNK, CHUNK)]], rows[b], gsem[b])
        cp.start()
        return cp

    def out_start(j):
        b = j % 2
        cp = pltpu.make_async_copy(
            rows[b], out_hbm.at[pl.ds(base + j * CHUNK, CHUNK)], osem[b])
        cp.start()
        return cp

    compute(0)
    g = {0: gather_start(0)}
    o = {}
    for j in range(NCHUNK):
        if j + 1 < NCHUNK:
            compute(j + 1)
            if j - 1 >= 0:
                o[j - 1].wait()
            g[j + 1] = gather_start(j + 1)
        g[j].wait()
        o[j] = out_start(j)
    o[NCHUNK - 2].wait()
    o[NCHUNK - 1].wait()


_gather_call = functools.partial(
    pl.kernel,
    out_type=jax.ShapeDtypeStruct((TOTAL, EMBED), jnp.float32),
    mesh=plsc.VectorSubcoreMesh(core_axis_name="c", subcore_axis_name="s"),
    compiler_params=pltpu.CompilerParams(use_tc_tiling_on_sc=False),
    scratch_types=[
        pltpu.VMEM((PER_W,), jnp.int32),
        pltpu.VMEM((CHUNK, EMBED), jnp.float32),
        pltpu.VMEM((CHUNK, EMBED), jnp.float32),
        pltpu.SemaphoreType.DMA,
        pltpu.SemaphoreType.DMA,
        pltpu.SemaphoreType.DMA,
        pltpu.SemaphoreType.DMA,
    ],
)(_body)


def kernel(x, tables):
    tab_flat = tables.reshape(N_FIELDS * VOCAB, EMBED)
    x_flat = x.reshape(TOTAL)
    out = _gather_call(tab_flat, x_flat)
    return out.reshape(BATCH, N_FIELDS * EMBED)


# speedup vs baseline: 1.2046x; 1.0008x over previous
import functools

import jax
import jax.numpy as jnp
from jax import lax
from jax.experimental import pallas as pl
from jax.experimental.pallas import tpu as pltpu
from jax.experimental.pallas import tpu_sc as plsc

N_FIELDS = 26
VOCAB = 100000
EMBED = 32
BATCH = 16384
TOTAL = BATCH * N_FIELDS

_INFO = plsc.get_sparse_core_info()
_NC = _INFO.num_cores
_NS = _INFO.num_subcores
NW = _NC * _NS
PER_W = TOTAL // NW
NSUB = 64
CS = PER_W // NSUB
NBUF = 8
LAG = NBUF - 1
NGROUP = NSUB // NBUF
LANES = 16


def _body(tab_hbm, x_hbm, out_hbm, idx_v, bufs, gsems, osems):
    wid = lax.axis_index("s") * _NC + lax.axis_index("c")
    base = wid * PER_W

    pltpu.sync_copy(x_hbm.at[pl.ds(base, PER_W)], idx_v)

    def compute(s):
        def step(k, carry):
            o = s * CS + k * LANES
            pos = o + lax.broadcasted_iota(jnp.int32, (LANES,), 0)
            off = lax.rem(pos, N_FIELDS) * VOCAB
            idx_v[pl.ds(o, LANES)] = idx_v[pl.ds(o, LANES)] + off
            return carry
        lax.fori_loop(0, CS // LANES, step, 0)

    def gather_start(s, b):
        pltpu.make_async_copy(
            tab_hbm.at[idx_v.at[pl.ds(s * CS, CS)]], bufs[b], gsems[b]).start()

    def gather_wait(b):
        pltpu.make_async_copy(
            tab_hbm.at[idx_v.at[pl.ds(0, CS)]], bufs[b], gsems[b]).wait()

    def out_start(s, b):
        pltpu.make_async_copy(
            bufs[b], out_hbm.at[pl.ds(base + s * CS, CS)], osems[b]).start()

    def out_wait(b):
        pltpu.make_async_copy(
            bufs[b], out_hbm.at[pl.ds(base, CS)], osems[b]).wait()

    def group(g, carry):
        for b in range(NBUF):
            s = g * NBUF + b

            @pl.when(g > 0)
            def _():
                out_wait(b)

            compute(s)
            gather_start(s, b)

            b2 = (b + 1) % NBUF

            @pl.when(s >= LAG)
            def _():
                gather_wait(b2)
                out_start(s - LAG, b2)

        return carry

    lax.fori_loop(0, NGROUP, group, 0)

    for s in range(NSUB - LAG, NSUB):
        b = s % NBUF
        gather_wait(b)
        out_start(s, b)
    for b in range(NBUF):
        out_wait(b)


def _flat_body(tab_hbm, x_hbm, out_hbm, idx_v,
               buf0, buf1, buf2, buf3, buf4, buf5, buf6, buf7,
               gs0, gs1, gs2, gs3, gs4, gs5, gs6, gs7,
               os0, os1, os2, os3, os4, os5, os6, os7):
    _body(tab_hbm, x_hbm, out_hbm, idx_v,
          (buf0, buf1, buf2, buf3, buf4, buf5, buf6, buf7),
          (gs0, gs1, gs2, gs3, gs4, gs5, gs6, gs7),
          (os0, os1, os2, os3, os4, os5, os6, os7))


_gather_call = functools.partial(
    pl.kernel,
    out_type=jax.ShapeDtypeStruct((TOTAL, EMBED), jnp.float32),
    mesh=plsc.VectorSubcoreMesh(core_axis_name="c", subcore_axis_name="s"),
    compiler_params=pltpu.CompilerParams(use_tc_tiling_on_sc=False),
    scratch_types=(
        [pltpu.VMEM((PER_W,), jnp.int32)]
        + [pltpu.VMEM((CS, EMBED), jnp.float32) for _ in range(NBUF)]
        + [pltpu.SemaphoreType.DMA for _ in range(2 * NBUF)]
    ),
)(_flat_body)


def kernel(x, tables):
    tab_flat = tables.reshape(N_FIELDS * VOCAB, EMBED)
    x_flat = x.reshape(TOTAL)
    out = _gather_call(tab_flat, x_flat)
    return out.reshape(BATCH, N_FIELDS * EMBED)


# speedup vs baseline: 3.7880x; 3.1445x over previous
import functools

import jax
import jax.numpy as jnp
from jax import lax
from jax.experimental import pallas as pl
from jax.experimental.pallas import tpu as pltpu
from jax.experimental.pallas import tpu_sc as plsc

N_FIELDS = 26
VOCAB = 100000
EMBED = 32
BATCH = 16384
NROWS = N_FIELDS * EMBED

_INFO = plsc.get_sparse_core_info()
_NC = _INFO.num_cores
_NS = _INFO.num_subcores
NW = _NC * _NS
LANES = 16
NB = 4096
NCHUNK = BATCH // NB


def _body(tab_hbm, x_hbm, out_hbm, line_v, xbuf, obuf, lsem, xsem, osem):
    w = lax.axis_index("s") * _NC + lax.axis_index("c")

    def task(f, carry):
        row = f * EMBED + w
        pltpu.sync_copy(tab_hbm.at[row, :], line_v)

        def chunk(c, carry2):
            n0 = c * NB
            pltpu.sync_copy(x_hbm.at[f, pl.ds(n0, NB)], xbuf)

            def step(j, carry3):
                s = j * LANES
                v = xbuf[pl.ds(s, LANES)]
                obuf[pl.ds(s, LANES)] = plsc.load_gather(line_v, [v])
                return carry3

            lax.fori_loop(0, NB // LANES, step, 0)
            pltpu.sync_copy(obuf, out_hbm.at[row, pl.ds(n0, NB)])
            return carry2

        lax.fori_loop(0, NCHUNK, chunk, 0)
        return carry

    lax.fori_loop(0, N_FIELDS, task, 0)


def _fixed_body(tab_hbm, x_hbm, out_hbm, line_v, xbuf, obuf, lsem, xsem, osem):
    _body(tab_hbm, x_hbm, out_hbm, line_v, xbuf, obuf, lsem, xsem, osem)


_gather_call = functools.partial(
    pl.kernel,
    out_type=jax.ShapeDtypeStruct((NROWS, BATCH), jnp.float32),
    mesh=plsc.VectorSubcoreMesh(core_axis_name="c", subcore_axis_name="s"),
    compiler_params=pltpu.CompilerParams(
        use_tc_tiling_on_sc=True, needs_layout_passes=False),
    scratch_types=[
        pltpu.VMEM((VOCAB,), jnp.float32),
        pltpu.VMEM((NB,), jnp.int32),
        pltpu.VMEM((NB,), jnp.float32),
        pltpu.SemaphoreType.DMA,
        pltpu.SemaphoreType.DMA,
        pltpu.SemaphoreType.DMA,
    ],
)(_fixed_body)


def kernel(x, tables):
    xt = jnp.swapaxes(x, 0, 1)
    tab2 = jnp.swapaxes(tables, 1, 2).reshape(NROWS, VOCAB)
    out_t = _gather_call(tab2, xt)
    return jnp.swapaxes(out_t, 0, 1)


# speedup vs baseline: 5.8321x; 1.5396x over previous
import functools

import jax
import jax.numpy as jnp
from jax import lax
from jax.experimental import pallas as pl
from jax.experimental.pallas import tpu as pltpu
from jax.experimental.pallas import tpu_sc as plsc

N_FIELDS = 26
VOCAB = 100000
EMBED = 32
BATCH = 16384
NROWS = N_FIELDS * EMBED

_INFO = plsc.get_sparse_core_info()
_NC = _INFO.num_cores
_NS = _INFO.num_subcores
NW = _NC * _NS
LANES = 16
NB = 4096
NCHUNK = BATCH // NB


def _body(tab_hbm, x_hbm, out_hbm, line_v, xb0, xb1, ob0, ob1,
          lsem, xsem0, xsem1, osem0, osem1):
    w = lax.axis_index("s") * _NC + lax.axis_index("c")
    xb = (xb0, xb1)
    ob = (ob0, ob1)
    xsem = (xsem0, xsem1)
    osem = (osem0, osem1)

    def line_start(row):
        pltpu.make_async_copy(tab_hbm.at[row, :], line_v, lsem).start()

    def line_wait():
        pltpu.make_async_copy(tab_hbm.at[0, :], line_v, lsem).wait()

    def x_start(f, c):
        pltpu.make_async_copy(
            x_hbm.at[f, pl.ds(c * NB, NB)], xb[c % 2], xsem[c % 2]).start()

    def x_wait(c):
        pltpu.make_async_copy(
            x_hbm.at[0, pl.ds(0, NB)], xb[c % 2], xsem[c % 2]).wait()

    def out_start(row, c):
        pltpu.make_async_copy(
            ob[c % 2], out_hbm.at[row, pl.ds(c * NB, NB)], osem[c % 2]).start()

    def out_wait(c):
        pltpu.make_async_copy(
            ob[c % 2], out_hbm.at[0, pl.ds(0, NB)], osem[c % 2]).wait()

    line_start(w)

    def task(f, carry):
        row = f * EMBED + w
        x_start(f, 0)
        line_wait()
        for c in range(NCHUNK):
            x_wait(c)
            if c + 1 < NCHUNK:
                x_start(f, c + 1)
            if c >= 2:
                out_wait(c - 2)
            xbuf = xb[c % 2]
            obuf = ob[c % 2]

            def step(j, carry3):
                s = j * (4 * LANES)
                for u in range(4):
                    su = s + u * LANES
                    v = xbuf[pl.ds(su, LANES)]
                    obuf[pl.ds(su, LANES)] = plsc.load_gather(line_v, [v])
                return carry3

            lax.fori_loop(0, NB // (4 * LANES), step, 0)
            out_start(row, c)
        @pl.when(f + 1 < N_FIELDS)
        def _():
            line_start((f + 1) * EMBED + w)

        out_wait(NCHUNK - 2)
        out_wait(NCHUNK - 1)
        return carry

    lax.fori_loop(0, N_FIELDS, task, 0)


def _fixed_body(tab_hbm, x_hbm, out_hbm, line_v, xb0, xb1, ob0, ob1,
                lsem, xsem0, xsem1, osem0, osem1):
    _body(tab_hbm, x_hbm, out_hbm, line_v, xb0, xb1, ob0, ob1,
          lsem, xsem0, xsem1, osem0, osem1)


_gather_call = functools.partial(
    pl.kernel,
    out_type=jax.ShapeDtypeStruct((NROWS, BATCH), jnp.float32),
    mesh=plsc.VectorSubcoreMesh(core_axis_name="c", subcore_axis_name="s"),
    compiler_params=pltpu.CompilerParams(
        use_tc_tiling_on_sc=True, needs_layout_passes=False),
    scratch_types=[
        pltpu.VMEM((VOCAB,), jnp.float32),
        pltpu.VMEM((NB,), jnp.int32),
        pltpu.VMEM((NB,), jnp.int32),
        pltpu.VMEM((NB,), jnp.float32),
        pltpu.VMEM((NB,), jnp.float32),
        pltpu.SemaphoreType.DMA,
        pltpu.SemaphoreType.DMA,
        pltpu.SemaphoreType.DMA,
        pltpu.SemaphoreType.DMA,
        pltpu.SemaphoreType.DMA,
    ],
)(_fixed_body)


def kernel(x, tables):
    xt = jnp.swapaxes(x, 0, 1)
    tab2 = jnp.swapaxes(tables, 1, 2).reshape(NROWS, VOCAB)
    out_t = _gather_call(tab2, xt)
    return jnp.swapaxes(out_t, 0, 1)


# speedup vs baseline: 6.5821x; 1.1286x over previous
import functools

import jax
import jax.numpy as jnp
from jax import lax
from jax.experimental import pallas as pl
from jax.experimental.pallas import tpu as pltpu
from jax.experimental.pallas import tpu_sc as plsc

N_FIELDS = 26
VOCAB = 100000
EMBED = 32
BATCH = 16384
NROWS = N_FIELDS * EMBED

_INFO = plsc.get_sparse_core_info()
_NC = _INFO.num_cores
_NS = _INFO.num_subcores
NW = _NC * _NS
LANES = 16
NB = 4096
NCHUNK = BATCH // NB


def _body(tab_hbm, x_hbm, out_hbm, line_v, xb0, xb1, ob0, ob1,
          lsem, xsem0, xsem1, osem0, osem1):
    w = lax.axis_index("s") * _NC + lax.axis_index("c")
    xb = (xb0, xb1)
    ob = (ob0, ob1)
    xsem = (xsem0, xsem1)
    osem = (osem0, osem1)

    def line_start(row):
        pltpu.make_async_copy(tab_hbm.at[row, :], line_v, lsem).start()

    def line_wait():
        pltpu.make_async_copy(tab_hbm.at[0, :], line_v, lsem).wait()

    def x_start(f, c):
        pltpu.make_async_copy(
            x_hbm.at[f, pl.ds(c * NB, NB)], xb[c % 2], xsem[c % 2]).start()

    def x_wait(c):
        pltpu.make_async_copy(
            x_hbm.at[0, pl.ds(0, NB)], xb[c % 2], xsem[c % 2]).wait()

    def out_start(row, c):
        pltpu.make_async_copy(
            ob[c % 2], out_hbm.at[row, pl.ds(c * NB, NB)], osem[c % 2]).start()

    def out_wait(c):
        pltpu.make_async_copy(
            ob[c % 2], out_hbm.at[0, pl.ds(0, NB)], osem[c % 2]).wait()

    line_start(w)

    def task(f, carry):
        row = f * EMBED + w
        x_start(f, 0)
        line_wait()
        for c in range(NCHUNK):
            x_wait(c)
            if c + 1 < NCHUNK:
                x_start(f, c + 1)
            if c >= 2:
                out_wait(c - 2)
            xbuf = xb[c % 2]
            obuf = ob[c % 2]

            @plsc.parallel_loop(0, NB // LANES, step=1, unroll=8)
            def _gather(j):
                s = j * LANES
                v = xbuf[pl.ds(s, LANES)]
                obuf[pl.ds(s, LANES)] = plsc.load_gather(line_v, [v])
            out_start(row, c)
        @pl.when(f + 1 < N_FIELDS)
        def _():
            line_start((f + 1) * EMBED + w)

        out_wait(NCHUNK - 2)
        out_wait(NCHUNK - 1)
        return carry

    lax.fori_loop(0, N_FIELDS, task, 0)


def _fixed_body(tab_hbm, x_hbm, out_hbm, line_v, xb0, xb1, ob0, ob1,
                lsem, xsem0, xsem1, osem0, osem1):
    _body(tab_hbm, x_hbm, out_hbm, line_v, xb0, xb1, ob0, ob1,
          lsem, xsem0, xsem1, osem0, osem1)


_gather_call = functools.partial(
    pl.kernel,
    out_type=jax.ShapeDtypeStruct((NROWS, BATCH), jnp.float32),
    mesh=plsc.VectorSubcoreMesh(core_axis_name="c", subcore_axis_name="s"),
    compiler_params=pltpu.CompilerParams(
        use_tc_tiling_on_sc=True, needs_layout_passes=False),
    scratch_types=[
        pltpu.VMEM((VOCAB,), jnp.float32),
        pltpu.VMEM((NB,), jnp.int32),
        pltpu.VMEM((NB,), jnp.int32),
        pltpu.VMEM((NB,), jnp.float32),
        pltpu.VMEM((NB,), jnp.float32),
        pltpu.SemaphoreType.DMA,
        pltpu.SemaphoreType.DMA,
        pltpu.SemaphoreType.DMA,
        pltpu.SemaphoreType.DMA,
        pltpu.SemaphoreType.DMA,
    ],
)(_fixed_body)


def kernel(x, tables):
    xt = jnp.swapaxes(x, 0, 1)
    tab2 = jnp.swapaxes(tables, 1, 2).reshape(NROWS, VOCAB)
    out_t = _gather_call(tab2, xt)
    return jnp.swapaxes(out_t, 0, 1)


# speedup vs baseline: 7.8583x; 1.1939x over previous
import functools

import jax
import jax.numpy as jnp
from jax import lax
from jax.experimental import pallas as pl
from jax.experimental.pallas import tpu as pltpu
from jax.experimental.pallas import tpu_sc as plsc

N_FIELDS = 26
VOCAB = 100000
EMBED = 32
BATCH = 16384
NROWS = N_FIELDS * EMBED

_INFO = plsc.get_sparse_core_info()
_NC = _INFO.num_cores
_NS = _INFO.num_subcores
NW = _NC * _NS
LANES = 16
NB = 4096
NCHUNK = BATCH // NB


NPAIR = 16
NTASK = 13


def _body(tab_hbm, x_hbm, out_hbm, line_v, xline, ob0, ob1,
          lsem, xsem, osem0, osem1):
    w = lax.axis_index("s") * _NC + lax.axis_index("c")
    ob = (ob0, ob1)
    osem = (osem0, osem1)

    def line_start(row):
        pltpu.make_async_copy(tab_hbm.at[row, :], line_v, lsem).start()

    def line_wait():
        pltpu.make_async_copy(tab_hbm.at[0, :], line_v, lsem).wait()

    def xline_start(f):
        pltpu.make_async_copy(x_hbm.at[f, :], xline, xsem).start()

    def xline_wait():
        pltpu.make_async_copy(x_hbm.at[0, :], xline, xsem).wait()

    def out_start(row, c):
        pltpu.make_async_copy(
            ob[c % 2], out_hbm.at[row, pl.ds(c * NB, NB)], osem[c % 2]).start()

    def out_wait(c):
        pltpu.make_async_copy(
            ob[c % 2], out_hbm.at[0, pl.ds(0, NB)], osem[c % 2]).wait()

    p = lax.rem(w, NPAIR)
    fbase = w // NPAIR

    line_start((fbase * EMBED) + 2 * p)
    xline_start(fbase)

    def gather_line(row):
        for c in range(NCHUNK):
            if c >= 2:
                out_wait(c - 2)
            obuf = ob[c % 2]
            cbase = c * NB

            @plsc.parallel_loop(0, NB // LANES, step=1, unroll=8)
            def _gather(j):
                s = j * LANES
                v = xline[pl.ds(cbase + s, LANES)]
                obuf[pl.ds(s, LANES)] = plsc.load_gather(line_v, [v])

            out_start(row, c)

    def tail_waits():
        out_wait(NCHUNK - 2)
        out_wait(NCHUNK - 1)

    def task(k, carry):
        f = fbase + 2 * k
        row0 = f * EMBED + 2 * p

        line_wait()
        xline_wait()
        gather_line(row0)
        line_start(row0 + 1)
        tail_waits()
        line_wait()
        gather_line(row0 + 1)

        @pl.when(k + 1 < NTASK)
        def _():
            line_start((fbase + 2 * (k + 1)) * EMBED + 2 * p)
            xline_start(fbase + 2 * (k + 1))

        tail_waits()
        return carry

    lax.fori_loop(0, NTASK, task, 0)


def _fixed_body(tab_hbm, x_hbm, out_hbm, line_v, xline, ob0, ob1,
                lsem, xsem, osem0, osem1):
    _body(tab_hbm, x_hbm, out_hbm, line_v, xline, ob0, ob1,
          lsem, xsem, osem0, osem1)


_gather_call = functools.partial(
    pl.kernel,
    out_type=jax.ShapeDtypeStruct((NROWS, BATCH), jnp.float32),
    mesh=plsc.VectorSubcoreMesh(core_axis_name="c", subcore_axis_name="s"),
    compiler_params=pltpu.CompilerParams(
        use_tc_tiling_on_sc=True, needs_layout_passes=False),
    scratch_types=[
        pltpu.VMEM((VOCAB,), jnp.float32),
        pltpu.VMEM((BATCH,), jnp.int32),
        pltpu.VMEM((NB,), jnp.float32),
        pltpu.VMEM((NB,), jnp.float32),
        pltpu.SemaphoreType.DMA,
        pltpu.SemaphoreType.DMA,
        pltpu.SemaphoreType.DMA,
        pltpu.SemaphoreType.DMA,
    ],
)(_fixed_body)


def kernel(x, tables):
    xt = jnp.swapaxes(x, 0, 1)
    tab2 = jnp.swapaxes(tables, 1, 2).reshape(NROWS, VOCAB)
    out_t = _gather_call(tab2, xt)
    return jnp.swapaxes(out_t, 0, 1)
